# Initial kernel scaffold; baseline (speedup 1.0000x reference)
import jax
import jax.numpy as jnp
from jax.experimental import pallas as pl


def kernel(x_init, emb, W, b, t_w, n_steps):
    raise NotImplementedError("write your pallas kernel here")


# speedup vs baseline: 1.6173x; 1.6173x over previous
import functools

import numpy as np
import jax
import jax.numpy as jnp
from jax.experimental import pallas as pl
from jax.experimental.pallas import tpu as pltpu

_TV = 2048
_B = 128

_TINY = np.float32(np.finfo(np.float32).tiny)


def _threefry2x32(k0, k1, x0, x1):
    def rotl(x, r):
        return (x << jnp.uint32(r)) | (x >> jnp.uint32(32 - r))

    ks2 = k0 ^ k1 ^ jnp.uint32(0x1BD11BDA)
    ks = (k0, k1, ks2)
    rots = ((13, 15, 26, 6), (17, 29, 16, 24))
    x0 = x0 + ks[0]
    x1 = x1 + ks[1]
    for i in range(5):
        for r in rots[i % 2]:
            x0 = x0 + x1
            x1 = rotl(x1, r)
            x1 = x1 ^ x0
        x0 = x0 + ks[(i + 1) % 3]
        x1 = x1 + ks[(i + 2) % 3] + jnp.uint32(i + 1)
    return x0, x1


def _sample_body(h_ref, w_ref, b_ref, kc_ref, x1_ref,
                 mx_ref, se_ref, bv_ref, bi_ref, *, V, T):
    p = pl.program_id(0)
    j = pl.program_id(1)
    logits = jnp.dot(h_ref[...], w_ref[...],
                     preferred_element_type=jnp.float32) + b_ref[...]
    col = j * _TV + jax.lax.broadcasted_iota(jnp.int32, (_B, _TV), 1)
    valid = col < V
    neg_inf = jnp.float32(-jnp.inf)

    @pl.when(p == 0)
    def _max_sum_pass():
        @pl.when(j == 0)
        def _init():
            mx_ref[...] = jnp.full((_B, 1), neg_inf, jnp.float32)
            se_ref[...] = jnp.zeros((_B, 1), jnp.float32)

        lm = jnp.where(valid, logits, neg_inf)
        tmax = jnp.max(lm, axis=1, keepdims=True)
        old_mx = mx_ref[...]
        new_mx = jnp.maximum(old_mx, tmax)
        e = jnp.where(valid, jnp.exp(logits - new_mx), jnp.float32(0.0))
        tsum = jnp.sum(e, axis=1, keepdims=True)
        se_ref[...] = se_ref[...] * jnp.exp(old_mx - new_mx) + tsum
        mx_ref[...] = new_mx

    @pl.when(p == 1)
    def _argmax_pass():
        @pl.when(j == 0)
        def _init():
            bv_ref[...] = jnp.full((_B, 1), neg_inf, jnp.float32)
            bi_ref[...] = jnp.zeros((_B, 1), jnp.int32)

        mx = mx_ref[...]
        se = se_ref[...]
        prob = jnp.exp(logits - mx) / se
        logp = jnp.log(prob + jnp.float32(1e-30))

        row = jax.lax.broadcasted_iota(jnp.int32, (_B, _TV), 0)
        cnt = (row * V + col).astype(jnp.uint32)
        k0 = kc_ref[0]
        k1 = kc_ref[1]
        o0, o1 = _threefry2x32(k0, k1, jnp.uint32(0), cnt)
        bits = o0 ^ o1
        fb = (bits >> jnp.uint32(9)) | jnp.uint32(0x3F800000)
        f = jax.lax.bitcast_convert_type(fb, jnp.float32) - jnp.float32(1.0)
        u = jnp.maximum(_TINY, f + _TINY)
        g = -jnp.log(-jnp.log(u))

        val = jnp.where(valid, g + logp, neg_inf)
        tmaxv = jnp.max(val, axis=1, keepdims=True)
        targ = (jnp.argmax(val, axis=1).astype(jnp.int32)
                .reshape(_B, 1) + j * _TV)
        better = tmaxv > bv_ref[...]
        bi_ref[...] = jnp.where(better, targ, bi_ref[...])
        bv_ref[...] = jnp.where(better, tmaxv, bv_ref[...])

        @pl.when(j == T - 1)
        def _emit():
            x1_ref[...] = bi_ref[...]


def _sample(hmat, w_pad, b_pad, kc, V):
    T = w_pad.shape[1] // _TV
    return pl.pallas_call(
        functools.partial(_sample_body, V=V, T=T),
        grid=(2, T),
        in_specs=[
            pl.BlockSpec((_B, hmat.shape[1]), lambda p, j: (0, 0)),
            pl.BlockSpec((w_pad.shape[0], _TV), lambda p, j: (0, j)),
            pl.BlockSpec((1, _TV), lambda p, j: (0, j)),
            pl.BlockSpec(memory_space=pltpu.SMEM),
        ],
        out_specs=pl.BlockSpec((_B, 1), lambda p, j: (0, 0)),
        out_shape=jax.ShapeDtypeStruct((_B, 1), jnp.int32),
        scratch_shapes=[
            pltpu.VMEM((_B, 1), jnp.float32),
            pltpu.VMEM((_B, 1), jnp.float32),
            pltpu.VMEM((_B, 1), jnp.float32),
            pltpu.VMEM((_B, 1), jnp.int32),
        ],
        compiler_params=pltpu.CompilerParams(
            dimension_semantics=("arbitrary", "arbitrary")),
    )(hmat, w_pad, b_pad, kc)


def kernel(x_init, emb, W, b, t_w, n_steps):
    B, S = x_init.shape
    V = W.shape[1]
    N = 10
    step_size = 1.0 / N
    t_disc = jnp.array([step_size * i for i in range(N)] + [1.0],
                       dtype=jnp.float32)
    t_disc = t_disc * (n_steps / N)

    T = -(-V // _TV)
    VP = T * _TV
    w_pad = jnp.pad(W, ((0, 0), (0, VP - V)))
    b_pad = jnp.pad(b, (0, VP - V)).reshape(1, VP)

    key = jax.random.key(42)
    x_t = x_init
    for i in range(N):
        t = t_disc[i]
        h = t_disc[i + 1] - t_disc[i]
        key, k_cat, k_jump, k_cat2 = jax.random.split(key, 4)
        hmat = jnp.take(emb, x_t[:, 0], axis=0) + t * t_w
        kc = jax.random.key_data(k_cat)
        x1 = _sample(hmat, w_pad, b_pad, kc, V)
        if i == N - 1:
            x_t = x1
        else:
            unif = jax.random.uniform(k_jump, (B, S))
            intensity = jnp.float32(1.0) / (1.0 - t)
            thr = 1.0 - jnp.exp(-h * intensity)
            mask = (unif < thr) & (x1 != x_t)
            x_t = jnp.where(mask, x1, x_t)
    return x_t


# speedup vs baseline: 1.7584x; 1.0872x over previous
import functools

import numpy as np
import jax
import jax.numpy as jnp
from jax.experimental import pallas as pl
from jax.experimental.pallas import tpu as pltpu

_TV = 2048
_B = 128

_TINY = np.float32(np.finfo(np.float32).tiny)


def _threefry2x32(k0, k1, x0, x1):
    def rotl(x, r):
        return (x << jnp.uint32(r)) | (x >> jnp.uint32(32 - r))

    ks2 = k0 ^ k1 ^ jnp.uint32(0x1BD11BDA)
    ks = (k0, k1, ks2)
    rots = ((13, 15, 26, 6), (17, 29, 16, 24))
    x0 = x0 + ks[0]
    x1 = x1 + ks[1]
    for i in range(5):
        for r in rots[i % 2]:
            x0 = x0 + x1
            x1 = rotl(x1, r)
            x1 = x1 ^ x0
        x0 = x0 + ks[(i + 1) % 3]
        x1 = x1 + ks[(i + 2) % 3] + jnp.uint32(i + 1)
    return x0, x1


def _pick_at(onehot, arr, neg_inf):
    return jnp.max(jnp.where(onehot, arr, neg_inf), axis=1, keepdims=True)


def _sample_body(h_ref, w_ref, b_ref, kc_ref, x1_ref,
                 mx_ref, se_ref,
                 s1_ref, g1_ref, l1_ref, i1_ref,
                 s2_ref, g2_ref, l2_ref, i2_ref, *, V, T):
    j = pl.program_id(0)
    neg_inf = jnp.float32(-jnp.inf)

    @pl.when(j == 0)
    def _init():
        mx_ref[...] = jnp.full((_B, 1), neg_inf, jnp.float32)
        se_ref[...] = jnp.zeros((_B, 1), jnp.float32)
        s1_ref[...] = jnp.full((_B, 1), neg_inf, jnp.float32)
        s2_ref[...] = jnp.full((_B, 1), neg_inf, jnp.float32)
        g1_ref[...] = jnp.zeros((_B, 1), jnp.float32)
        g2_ref[...] = jnp.zeros((_B, 1), jnp.float32)
        l1_ref[...] = jnp.zeros((_B, 1), jnp.float32)
        l2_ref[...] = jnp.zeros((_B, 1), jnp.float32)
        i1_ref[...] = jnp.zeros((_B, 1), jnp.int32)
        i2_ref[...] = jnp.zeros((_B, 1), jnp.int32)

    @pl.when(j < T)
    def _sweep():
        logits = jnp.dot(h_ref[...], w_ref[...],
                         preferred_element_type=jnp.float32) + b_ref[...]
        lane = jax.lax.broadcasted_iota(jnp.int32, (_B, _TV), 1)
        col = j * _TV + lane
        valid = col < V

        lm = jnp.where(valid, logits, neg_inf)
        tmax = jnp.max(lm, axis=1, keepdims=True)
        old_mx = mx_ref[...]
        new_mx = jnp.maximum(old_mx, tmax)
        e = jnp.where(valid, jnp.exp(logits - new_mx), jnp.float32(0.0))
        se_ref[...] = se_ref[...] * jnp.exp(old_mx - new_mx) \
            + jnp.sum(e, axis=1, keepdims=True)
        mx_ref[...] = new_mx

        row = jax.lax.broadcasted_iota(jnp.int32, (_B, _TV), 0)
        cnt = (row * V + col).astype(jnp.uint32)
        o0, o1 = _threefry2x32(kc_ref[0], kc_ref[1], jnp.uint32(0), cnt)
        bits = o0 ^ o1
        fb = (bits >> jnp.uint32(9)) | jnp.uint32(0x3F800000)
        f = jax.lax.bitcast_convert_type(fb, jnp.float32) - jnp.float32(1.0)
        u = jnp.maximum(_TINY, f + _TINY)
        g = -jnp.log(-jnp.log(u))

        s = jnp.where(valid, g + logits, neg_inf)
        m1 = jnp.max(s, axis=1, keepdims=True)
        a1 = jnp.argmax(s, axis=1).astype(jnp.int32).reshape(_B, 1)
        oh1 = lane == a1
        lt1 = _pick_at(oh1, logits, neg_inf)
        gt1 = _pick_at(oh1, g, neg_inf)
        s_m = jnp.where(oh1, neg_inf, s)
        m2 = jnp.max(s_m, axis=1, keepdims=True)
        a2 = jnp.argmax(s_m, axis=1).astype(jnp.int32).reshape(_B, 1)
        oh2 = lane == a2
        lt2 = _pick_at(oh2, logits, neg_inf)
        gt2 = _pick_at(oh2, g, neg_inf)
        ga1 = a1 + j * _TV
        ga2 = a2 + j * _TV

        rs1, rs2 = s1_ref[...], s2_ref[...]
        take1 = m1 > rs1
        n1_s = jnp.where(take1, m1, rs1)
        n1_g = jnp.where(take1, gt1, g1_ref[...])
        n1_l = jnp.where(take1, lt1, l1_ref[...])
        n1_i = jnp.where(take1, ga1, i1_ref[...])
        c_s = jnp.where(take1, rs1, rs2)
        c_g = jnp.where(take1, g1_ref[...], g2_ref[...])
        c_l = jnp.where(take1, l1_ref[...], l2_ref[...])
        c_i = jnp.where(take1, i1_ref[...], i2_ref[...])
        d_s = jnp.where(take1, m2, m1)
        d_g = jnp.where(take1, gt2, gt1)
        d_l = jnp.where(take1, lt2, lt1)
        d_i = jnp.where(take1, ga2, ga1)
        take2 = d_s > c_s
        s2_ref[...] = jnp.where(take2, d_s, c_s)
        g2_ref[...] = jnp.where(take2, d_g, c_g)
        l2_ref[...] = jnp.where(take2, d_l, c_l)
        i2_ref[...] = jnp.where(take2, d_i, c_i)
        s1_ref[...] = n1_s
        g1_ref[...] = n1_g
        l1_ref[...] = n1_l
        i1_ref[...] = n1_i

    @pl.when(j == T)
    def _finalize():
        mx = mx_ref[...]
        se = se_ref[...]
        eps = jnp.float32(1e-30)
        v1 = jnp.log(jnp.exp(l1_ref[...] - mx) / se + eps) + g1_ref[...]
        v2 = jnp.log(jnp.exp(l2_ref[...] - mx) / se + eps) + g2_ref[...]
        i1 = i1_ref[...]
        i2 = i2_ref[...]
        pick2 = (v2 > v1) | ((v2 == v1) & (i2 < i1))
        x1_ref[...] = jnp.where(pick2, i2, i1)


def _sample(hmat, w_pad, b_pad, kc, V):
    T = w_pad.shape[1] // _TV
    f32 = jnp.float32
    return pl.pallas_call(
        functools.partial(_sample_body, V=V, T=T),
        grid=(T + 1,),
        in_specs=[
            pl.BlockSpec((_B, hmat.shape[1]), lambda j: (0, 0)),
            pl.BlockSpec((w_pad.shape[0], _TV), lambda j: (0, jnp.minimum(j, T - 1))),
            pl.BlockSpec((1, _TV), lambda j: (0, jnp.minimum(j, T - 1))),
            pl.BlockSpec(memory_space=pltpu.SMEM),
        ],
        out_specs=pl.BlockSpec((_B, 1), lambda j: (0, 0)),
        out_shape=jax.ShapeDtypeStruct((_B, 1), jnp.int32),
        scratch_shapes=[
            pltpu.VMEM((_B, 1), f32), pltpu.VMEM((_B, 1), f32),
            pltpu.VMEM((_B, 1), f32), pltpu.VMEM((_B, 1), f32),
            pltpu.VMEM((_B, 1), f32), pltpu.VMEM((_B, 1), jnp.int32),
            pltpu.VMEM((_B, 1), f32), pltpu.VMEM((_B, 1), f32),
            pltpu.VMEM((_B, 1), f32), pltpu.VMEM((_B, 1), jnp.int32),
        ],
        compiler_params=pltpu.CompilerParams(
            dimension_semantics=("arbitrary",)),
    )(hmat, w_pad, b_pad, kc)


def kernel(x_init, emb, W, b, t_w, n_steps):
    B, S = x_init.shape
    V = W.shape[1]
    N = 10
    step_size = 1.0 / N
    t_disc = jnp.array([step_size * i for i in range(N)] + [1.0],
                       dtype=jnp.float32)
    t_disc = t_disc * (n_steps / N)

    T = -(-V // _TV)
    VP = T * _TV
    w_pad = jnp.pad(W, ((0, 0), (0, VP - V)))
    b_pad = jnp.pad(b, (0, VP - V)).reshape(1, VP)

    key = jax.random.key(42)
    x_t = x_init
    for i in range(N):
        t = t_disc[i]
        h = t_disc[i + 1] - t_disc[i]
        key, k_cat, k_jump, k_cat2 = jax.random.split(key, 4)
        hmat = jnp.take(emb, x_t[:, 0], axis=0) + t * t_w
        kc = jax.random.key_data(k_cat)
        x1 = _sample(hmat, w_pad, b_pad, kc, V)
        if i == N - 1:
            x_t = x1
        else:
            unif = jax.random.uniform(k_jump, (B, S))
            intensity = jnp.float32(1.0) / (1.0 - t)
            thr = 1.0 - jnp.exp(-h * intensity)
            mask = (unif < thr) & (x1 != x_t)
            x_t = jnp.where(mask, x1, x_t)
    return x_t


# speedup vs baseline: 1.8839x; 1.0714x over previous
import functools

import numpy as np
import jax
import jax.numpy as jnp
from jax.experimental import pallas as pl
from jax.experimental.pallas import tpu as pltpu

_TV = 2048
_B = 128
_N = 10

_TINY = np.float32(np.finfo(np.float32).tiny)


def _threefry2x32(k0, k1, x0, x1):
    def rotl(x, r):
        return (x << jnp.uint32(r)) | (x >> jnp.uint32(32 - r))

    ks2 = k0 ^ k1 ^ jnp.uint32(0x1BD11BDA)
    ks = (k0, k1, ks2)
    rots = ((13, 15, 26, 6), (17, 29, 16, 24))
    x0 = x0 + ks[0]
    x1 = x1 + ks[1]
    for i in range(5):
        for r in rots[i % 2]:
            x0 = x0 + x1
            x1 = rotl(x1, r)
            x1 = x1 ^ x0
        x0 = x0 + ks[(i + 1) % 3]
        x1 = x1 + ks[(i + 2) % 3] + jnp.uint32(i + 1)
    return x0, x1


def _gumbel_at(k0, k1, cnt_u32):
    o0, o1 = _threefry2x32(k0, k1, jnp.uint32(0), cnt_u32)
    bits = o0 ^ o1
    fb = (bits >> jnp.uint32(9)) | jnp.uint32(0x3F800000)
    f = jax.lax.bitcast_convert_type(fb, jnp.float32) - jnp.float32(1.0)
    u = jnp.maximum(_TINY, f + _TINY)
    return -jnp.log(-jnp.log(u))


def _pick_at(onehot, arr, neg_inf):
    return jnp.max(jnp.where(onehot, arr, neg_inf), axis=1, keepdims=True)


def _body(xinit_ref, ub_ref, thrb_ref, keys_ref,
          emb_ref, w_ref, b_ref, tv_ref,
          out_ref,
          mx_ref, se_ref, s1_ref, l1_ref, i1_ref, s2_ref, l2_ref, i2_ref,
          h_ref, xs_ref, x1v_ref, x1s_ref, sem, *, V, T):
    i = pl.program_id(0)
    j = pl.program_id(1)
    neg_inf = jnp.float32(-jnp.inf)

    @pl.when(j == 0)
    def _prologue():
        tv = tv_ref[0]

        @pl.when(i == 0)
        def _first():
            def loop0(b, _):
                x = xinit_ref[b]
                xs_ref[b, 0] = x
                h_ref[pl.ds(b, 1), :] = emb_ref[pl.ds(x, 1), :] + tv
                return _
            jax.lax.fori_loop(0, _B, loop0, None)

        @pl.when(i > 0)
        def _update():
            def loop(b, _):
                xp = xs_ref[b, 0]
                x1 = x1s_ref[b, 0]
                jump = jnp.logical_and(ub_ref[i - 1, b] < thrb_ref[i - 1],
                                       x1 != xp)
                x = jnp.where(jump, x1, xp)
                xs_ref[b, 0] = x
                h_ref[pl.ds(b, 1), :] = emb_ref[pl.ds(x, 1), :] + tv
                return _
            jax.lax.fori_loop(0, _B, loop, None)

        mx_ref[...] = jnp.full((_B, 1), neg_inf, jnp.float32)
        se_ref[...] = jnp.zeros((_B, 1), jnp.float32)
        s1_ref[...] = jnp.full((_B, 1), neg_inf, jnp.float32)
        s2_ref[...] = jnp.full((_B, 1), neg_inf, jnp.float32)
        l1_ref[...] = jnp.zeros((_B, 1), jnp.float32)
        l2_ref[...] = jnp.zeros((_B, 1), jnp.float32)
        i1_ref[...] = jnp.zeros((_B, 1), jnp.int32)
        i2_ref[...] = jnp.zeros((_B, 1), jnp.int32)

    @pl.when(j < T)
    def _sweep():
        logits = jnp.dot(h_ref[...], w_ref[...],
                         preferred_element_type=jnp.float32) + b_ref[...]
        lane = jax.lax.broadcasted_iota(jnp.int32, (_B, _TV), 1)
        col = j * _TV + lane
        valid = col < V

        lm = jnp.where(valid, logits, neg_inf)
        tmax = jnp.max(lm, axis=1, keepdims=True)
        old_mx = mx_ref[...]
        new_mx = jnp.maximum(old_mx, tmax)
        e = jnp.where(valid, jnp.exp(logits - new_mx), jnp.float32(0.0))
        se_ref[...] = se_ref[...] * jnp.exp(old_mx - new_mx) \
            + jnp.sum(e, axis=1, keepdims=True)
        mx_ref[...] = new_mx

        row = jax.lax.broadcasted_iota(jnp.int32, (_B, _TV), 0)
        cnt = (row * V + col).astype(jnp.uint32)
        g = _gumbel_at(keys_ref[i, 0], keys_ref[i, 1], cnt)

        s = jnp.where(valid, g + logits, neg_inf)
        m1 = jnp.max(s, axis=1, keepdims=True)
        a1 = jnp.argmax(s, axis=1).astype(jnp.int32).reshape(_B, 1)
        oh1 = lane == a1
        lt1 = _pick_at(oh1, logits, neg_inf)
        s_m = jnp.where(oh1, neg_inf, s)
        m2 = jnp.max(s_m, axis=1, keepdims=True)
        a2 = jnp.argmax(s_m, axis=1).astype(jnp.int32).reshape(_B, 1)
        lt2 = _pick_at(lane == a2, logits, neg_inf)
        ga1 = a1 + j * _TV
        ga2 = a2 + j * _TV

        rs1, rs2 = s1_ref[...], s2_ref[...]
        take1 = m1 > rs1
        n1_s = jnp.where(take1, m1, rs1)
        n1_l = jnp.where(take1, lt1, l1_ref[...])
        n1_i = jnp.where(take1, ga1, i1_ref[...])
        c_s = jnp.where(take1, rs1, rs2)
        c_l = jnp.where(take1, l1_ref[...], l2_ref[...])
        c_i = jnp.where(take1, i1_ref[...], i2_ref[...])
        d_s = jnp.where(take1, m2, m1)
        d_l = jnp.where(take1, lt2, lt1)
        d_i = jnp.where(take1, ga2, ga1)
        take2 = d_s > c_s
        s2_ref[...] = jnp.where(take2, d_s, c_s)
        l2_ref[...] = jnp.where(take2, d_l, c_l)
        i2_ref[...] = jnp.where(take2, d_i, c_i)
        s1_ref[...] = n1_s
        l1_ref[...] = n1_l
        i1_ref[...] = n1_i

    @pl.when(j == T)
    def _finalize():
        mx = mx_ref[...]
        se = se_ref[...]
        eps = jnp.float32(1e-30)
        rowv = jax.lax.broadcasted_iota(jnp.int32, (_B, 1), 0)
        i1 = i1_ref[...]
        i2 = i2_ref[...]
        k0 = keys_ref[i, 0]
        k1 = keys_ref[i, 1]
        g1 = _gumbel_at(k0, k1, (rowv * V + i1).astype(jnp.uint32))
        g2 = _gumbel_at(k0, k1, (rowv * V + i2).astype(jnp.uint32))
        v1 = jnp.log(jnp.exp(l1_ref[...] - mx) / se + eps) + g1
        v2 = jnp.log(jnp.exp(l2_ref[...] - mx) / se + eps) + g2
        pick2 = (v2 > v1) | ((v2 == v1) & (i2 < i1))
        x1 = jnp.where(pick2, i2, i1)

        @pl.when(i < _N - 1)
        def _handoff():
            x1v_ref[...] = x1
            dma = pltpu.make_async_copy(x1v_ref, x1s_ref, sem)
            dma.start()
            dma.wait()

        @pl.when(i == _N - 1)
        def _emit():
            out_ref[...] = x1


def kernel(x_init, emb, W, b, t_w, n_steps):
    B, S = x_init.shape
    V, D = emb.shape
    step_size = 1.0 / _N
    t_disc = jnp.array([step_size * i for i in range(_N)] + [1.0],
                       dtype=jnp.float32)
    t_disc = t_disc * (n_steps / _N)

    T = -(-V // _TV)
    VP = T * _TV
    w_pad = jnp.pad(W, ((0, 0), (0, VP - V)))
    b_pad = jnp.pad(b, (0, VP - V)).reshape(1, VP)

    key = jax.random.key(42)
    keys = []
    unifs = []
    thrs = []
    for i in range(_N):
        key, k_cat, k_jump, k_cat2 = jax.random.split(key, 4)
        keys.append(jax.random.key_data(k_cat))
        if i < _N - 1:
            t = t_disc[i]
            h = t_disc[i + 1] - t_disc[i]
            unifs.append(jax.random.uniform(k_jump, (B, S)).reshape(B))
            intensity = jnp.float32(1.0) / (1.0 - t)
            thrs.append(1.0 - jnp.exp(-h * intensity))
    keys = jnp.stack(keys)
    ub = jax.lax.bitcast_convert_type(
        jnp.stack(unifs + [jnp.zeros(B, jnp.float32)]), jnp.int32)
    thrb = jax.lax.bitcast_convert_type(
        jnp.stack(thrs + [jnp.float32(0.0)]), jnp.int32)
    tvec = (t_disc[:_N, None] * t_w[None, :]).reshape(_N, 1, D)

    f32 = jnp.float32
    out = pl.pallas_call(
        functools.partial(_body, V=V, T=T),
        grid=(_N, T + 1),
        in_specs=[
            pl.BlockSpec(memory_space=pltpu.SMEM),
            pl.BlockSpec(memory_space=pltpu.SMEM),
            pl.BlockSpec(memory_space=pltpu.SMEM),
            pl.BlockSpec(memory_space=pltpu.SMEM),
            pl.BlockSpec((V, D), lambda i, j: (0, 0)),
            pl.BlockSpec((D, _TV),
                         lambda i, j: (0, jnp.minimum(j, T - 1))),
            pl.BlockSpec((1, _TV),
                         lambda i, j: (0, jnp.minimum(j, T - 1))),
            pl.BlockSpec((1, 1, D), lambda i, j: (i, 0, 0)),
        ],
        out_specs=pl.BlockSpec((_B, 1), lambda i, j: (0, 0)),
        out_shape=jax.ShapeDtypeStruct((_B, 1), jnp.int32),
        scratch_shapes=[
            pltpu.VMEM((_B, 1), f32), pltpu.VMEM((_B, 1), f32),
            pltpu.VMEM((_B, 1), f32), pltpu.VMEM((_B, 1), f32),
            pltpu.VMEM((_B, 1), jnp.int32),
            pltpu.VMEM((_B, 1), f32), pltpu.VMEM((_B, 1), f32),
            pltpu.VMEM((_B, 1), jnp.int32),
            pltpu.VMEM((_B, D), f32),
            pltpu.SMEM((_B, 1), jnp.int32),
            pltpu.VMEM((_B, 1), jnp.int32),
            pltpu.SMEM((_B, 1), jnp.int32),
            pltpu.SemaphoreType.DMA,
        ],
        compiler_params=pltpu.CompilerParams(
            dimension_semantics=("arbitrary", "arbitrary"),
            vmem_limit_bytes=60 * 1024 * 1024),
    )(x_init[:, 0], ub, thrb, keys, emb, w_pad, b_pad, tvec)
    return out


# speedup vs baseline: 1.9116x; 1.0147x over previous
import functools

import numpy as np
import jax
import jax.numpy as jnp
from jax.experimental import pallas as pl
from jax.experimental.pallas import tpu as pltpu

_TV = 2048
_B = 128
_N = 10

_TINY = np.float32(np.finfo(np.float32).tiny)


def _threefry2x32(k0, k1, x0, x1):
    def rotl(x, r):
        return (x << jnp.uint32(r)) | (x >> jnp.uint32(32 - r))

    ks2 = k0 ^ k1 ^ jnp.uint32(0x1BD11BDA)
    ks = (k0, k1, ks2)
    rots = ((13, 15, 26, 6), (17, 29, 16, 24))
    x0 = x0 + ks[0]
    x1 = x1 + ks[1]
    for i in range(5):
        for r in rots[i % 2]:
            x0 = x0 + x1
            x1 = rotl(x1, r)
            x1 = x1 ^ x0
        x0 = x0 + ks[(i + 1) % 3]
        x1 = x1 + ks[(i + 2) % 3] + jnp.uint32(i + 1)
    return x0, x1


def _gumbel_at(k0, k1, cnt_u32):
    o0, o1 = _threefry2x32(k0, k1, jnp.uint32(0), cnt_u32)
    bits = o0 ^ o1
    fb = (bits >> jnp.uint32(9)) | jnp.uint32(0x3F800000)
    f = jax.lax.bitcast_convert_type(fb, jnp.float32) - jnp.float32(1.0)
    u = jnp.maximum(_TINY, f + _TINY)
    return -jnp.log(-jnp.log(u))


def _pick_at(onehot, arr, neg_inf):
    return jnp.max(jnp.where(onehot, arr, neg_inf), axis=1, keepdims=True)


def _body(xinit_ref, ub_ref, thrb_ref, keys_ref,
          emb_ref, w_ref, b_ref, tv_ref,
          out_ref,
          mx_ref, se_ref, s1_ref, l1_ref, i1_ref, s2_ref, l2_ref, i2_ref,
          h_ref, xs_ref, x1v_ref, x1s_ref, cnt0_ref, sem, *, V, T):
    i = pl.program_id(0)
    j = pl.program_id(1)
    neg_inf = jnp.float32(-jnp.inf)

    @pl.when(j == 0)
    def _prologue():
        tv = tv_ref[0]

        @pl.when(i == 0)
        def _first():
            row = jax.lax.broadcasted_iota(jnp.int32, (_B, _TV), 0)
            lane = jax.lax.broadcasted_iota(jnp.int32, (_B, _TV), 1)
            cnt0_ref[...] = row * V + lane

            def loop0(b, _):
                x = xinit_ref[b]
                xs_ref[b, 0] = x
                h_ref[pl.ds(b, 1), :] = emb_ref[pl.ds(x, 1), :] + tv
                return _
            jax.lax.fori_loop(0, _B, loop0, None)

        @pl.when(i > 0)
        def _update():
            def loop(b, _):
                xp = xs_ref[b, 0]
                x1 = x1s_ref[b, 0]
                jump = jnp.logical_and(ub_ref[i - 1, b] < thrb_ref[i - 1],
                                       x1 != xp)
                x = jnp.where(jump, x1, xp)
                xs_ref[b, 0] = x
                h_ref[pl.ds(b, 1), :] = emb_ref[pl.ds(x, 1), :] + tv
                return _
            jax.lax.fori_loop(0, _B, loop, None)

        mx_ref[...] = jnp.full((_B, 1), neg_inf, jnp.float32)
        se_ref[...] = jnp.zeros((_B, 1), jnp.float32)
        s1_ref[...] = jnp.full((_B, 1), neg_inf, jnp.float32)
        s2_ref[...] = jnp.full((_B, 1), neg_inf, jnp.float32)
        l1_ref[...] = jnp.zeros((_B, 1), jnp.float32)
        l2_ref[...] = jnp.zeros((_B, 1), jnp.float32)
        i1_ref[...] = jnp.zeros((_B, 1), jnp.int32)
        i2_ref[...] = jnp.zeros((_B, 1), jnp.int32)

    @pl.when(j < T)
    def _sweep():
        logits = jnp.dot(h_ref[...], w_ref[...],
                         preferred_element_type=jnp.float32) + b_ref[...]

        tmax = jnp.max(logits, axis=1, keepdims=True)
        old_mx = mx_ref[...]
        new_mx = jnp.maximum(old_mx, tmax)
        e = jnp.exp(logits - new_mx)
        se_ref[...] = se_ref[...] * jnp.exp(old_mx - new_mx) \
            + jnp.sum(e, axis=1, keepdims=True)
        mx_ref[...] = new_mx

        cnt0 = cnt0_ref[...]
        cnt = (cnt0 + j * _TV).astype(jnp.uint32)
        g = _gumbel_at(keys_ref[i, 0], keys_ref[i, 1], cnt)

        s = g + logits
        rowv = jax.lax.broadcasted_iota(jnp.int32, (_B, 1), 0) * V
        m1 = jnp.max(s, axis=1, keepdims=True)
        a1 = jnp.argmax(s, axis=1).astype(jnp.int32).reshape(_B, 1)
        oh1 = cnt0 == rowv + a1
        lt1 = _pick_at(oh1, logits, neg_inf)
        s_m = jnp.where(oh1, neg_inf, s)
        m2 = jnp.max(s_m, axis=1, keepdims=True)
        a2 = jnp.argmax(s_m, axis=1).astype(jnp.int32).reshape(_B, 1)
        lt2 = _pick_at(cnt0 == rowv + a2, logits, neg_inf)
        ga1 = a1 + j * _TV
        ga2 = a2 + j * _TV

        rs1, rs2 = s1_ref[...], s2_ref[...]
        take1 = m1 > rs1
        n1_s = jnp.where(take1, m1, rs1)
        n1_l = jnp.where(take1, lt1, l1_ref[...])
        n1_i = jnp.where(take1, ga1, i1_ref[...])
        c_s = jnp.where(take1, rs1, rs2)
        c_l = jnp.where(take1, l1_ref[...], l2_ref[...])
        c_i = jnp.where(take1, i1_ref[...], i2_ref[...])
        d_s = jnp.where(take1, m2, m1)
        d_l = jnp.where(take1, lt2, lt1)
        d_i = jnp.where(take1, ga2, ga1)
        take2 = d_s > c_s
        s2_ref[...] = jnp.where(take2, d_s, c_s)
        l2_ref[...] = jnp.where(take2, d_l, c_l)
        i2_ref[...] = jnp.where(take2, d_i, c_i)
        s1_ref[...] = n1_s
        l1_ref[...] = n1_l
        i1_ref[...] = n1_i

    @pl.when(j == T)
    def _finalize():
        mx = mx_ref[...]
        se = se_ref[...]
        eps = jnp.float32(1e-30)
        rowv = jax.lax.broadcasted_iota(jnp.int32, (_B, 1), 0)
        i1 = i1_ref[...]
        i2 = i2_ref[...]
        k0 = keys_ref[i, 0]
        k1 = keys_ref[i, 1]
        g1 = _gumbel_at(k0, k1, (rowv * V + i1).astype(jnp.uint32))
        g2 = _gumbel_at(k0, k1, (rowv * V + i2).astype(jnp.uint32))
        v1 = jnp.log(jnp.exp(l1_ref[...] - mx) / se + eps) + g1
        v2 = jnp.log(jnp.exp(l2_ref[...] - mx) / se + eps) + g2
        pick2 = (v2 > v1) | ((v2 == v1) & (i2 < i1))
        x1 = jnp.where(pick2, i2, i1)

        @pl.when(i < _N - 1)
        def _handoff():
            x1v_ref[...] = x1
            dma = pltpu.make_async_copy(x1v_ref, x1s_ref, sem)
            dma.start()
            dma.wait()

        @pl.when(i == _N - 1)
        def _emit():
            out_ref[...] = x1


def kernel(x_init, emb, W, b, t_w, n_steps):
    B, S = x_init.shape
    V, D = emb.shape
    step_size = 1.0 / _N
    t_disc = jnp.array([step_size * i for i in range(_N)] + [1.0],
                       dtype=jnp.float32)
    t_disc = t_disc * (n_steps / _N)

    T = -(-V // _TV)
    VP = T * _TV
    w_pad = jnp.pad(W, ((0, 0), (0, VP - V)))
    b_pad = jnp.pad(b, (0, VP - V),
                    constant_values=-jnp.inf).reshape(1, VP)

    key = jax.random.key(42)
    keys = []
    unifs = []
    thrs = []
    for i in range(_N):
        key, k_cat, k_jump, k_cat2 = jax.random.split(key, 4)
        keys.append(jax.random.key_data(k_cat))
        if i < _N - 1:
            t = t_disc[i]
            h = t_disc[i + 1] - t_disc[i]
            unifs.append(jax.random.uniform(k_jump, (B, S)).reshape(B))
            intensity = jnp.float32(1.0) / (1.0 - t)
            thrs.append(1.0 - jnp.exp(-h * intensity))
    keys = jnp.stack(keys)
    ub = jax.lax.bitcast_convert_type(
        jnp.stack(unifs + [jnp.zeros(B, jnp.float32)]), jnp.int32)
    thrb = jax.lax.bitcast_convert_type(
        jnp.stack(thrs + [jnp.float32(0.0)]), jnp.int32)
    tvec = (t_disc[:_N, None] * t_w[None, :]).reshape(_N, 1, D)

    f32 = jnp.float32
    out = pl.pallas_call(
        functools.partial(_body, V=V, T=T),
        grid=(_N, T + 1),
        in_specs=[
            pl.BlockSpec(memory_space=pltpu.SMEM),
            pl.BlockSpec(memory_space=pltpu.SMEM),
            pl.BlockSpec(memory_space=pltpu.SMEM),
            pl.BlockSpec(memory_space=pltpu.SMEM),
            pl.BlockSpec((V, D), lambda i, j: (0, 0)),
            pl.BlockSpec((D, _TV),
                         lambda i, j: (0, jnp.minimum(j, T - 1))),
            pl.BlockSpec((1, _TV),
                         lambda i, j: (0, jnp.minimum(j, T - 1))),
            pl.BlockSpec((1, 1, D), lambda i, j: (i, 0, 0)),
        ],
        out_specs=pl.BlockSpec((_B, 1), lambda i, j: (0, 0)),
        out_shape=jax.ShapeDtypeStruct((_B, 1), jnp.int32),
        scratch_shapes=[
            pltpu.VMEM((_B, 1), f32), pltpu.VMEM((_B, 1), f32),
            pltpu.VMEM((_B, 1), f32), pltpu.VMEM((_B, 1), f32),
            pltpu.VMEM((_B, 1), jnp.int32),
            pltpu.VMEM((_B, 1), f32), pltpu.VMEM((_B, 1), f32),
            pltpu.VMEM((_B, 1), jnp.int32),
            pltpu.VMEM((_B, D), f32),
            pltpu.SMEM((_B, 1), jnp.int32),
            pltpu.VMEM((_B, 1), jnp.int32),
            pltpu.SMEM((_B, 1), jnp.int32),
            pltpu.VMEM((_B, _TV), jnp.int32),
            pltpu.SemaphoreType.DMA,
        ],
        compiler_params=pltpu.CompilerParams(
            dimension_semantics=("arbitrary", "arbitrary"),
            vmem_limit_bytes=60 * 1024 * 1024),
    )(x_init[:, 0], ub, thrb, keys, emb, w_pad, b_pad, tvec)
    return out


# speedup vs baseline: 1.9372x; 1.0134x over previous
import functools

import numpy as np
import jax
import jax.numpy as jnp
from jax.experimental import pallas as pl
from jax.experimental.pallas import tpu as pltpu

_TV = 4096
_B = 128
_N = 10

_TINY = np.float32(np.finfo(np.float32).tiny)


def _threefry2x32(k0, k1, x0, x1):
    def rotl(x, r):
        return (x << jnp.uint32(r)) | (x >> jnp.uint32(32 - r))

    ks2 = k0 ^ k1 ^ jnp.uint32(0x1BD11BDA)
    ks = (k0, k1, ks2)
    rots = ((13, 15, 26, 6), (17, 29, 16, 24))
    x0 = x0 + ks[0]
    x1 = x1 + ks[1]
    for i in range(5):
        for r in rots[i % 2]:
            x0 = x0 + x1
            x1 = rotl(x1, r)
            x1 = x1 ^ x0
        x0 = x0 + ks[(i + 1) % 3]
        x1 = x1 + ks[(i + 2) % 3] + jnp.uint32(i + 1)
    return x0, x1


def _gumbel_at(k0, k1, cnt_u32):
    o0, o1 = _threefry2x32(k0, k1, jnp.uint32(0), cnt_u32)
    bits = o0 ^ o1
    fb = (bits >> jnp.uint32(9)) | jnp.uint32(0x3F800000)
    f = jax.lax.bitcast_convert_type(fb, jnp.float32) - jnp.float32(1.0)
    u = jnp.maximum(_TINY, f + _TINY)
    return -jnp.log(-jnp.log(u))


def _pick_at(onehot, arr, neg_inf):
    return jnp.max(jnp.where(onehot, arr, neg_inf), axis=1, keepdims=True)


def _body(xinit_ref, ub_ref, thrb_ref, keys_ref,
          emb_ref, w_ref, b_ref, tv_ref,
          out_ref,
          mx_ref, se_ref, s1_ref, l1_ref, i1_ref, s2_ref, l2_ref, i2_ref,
          h_ref, xs_ref, x1v_ref, x1s_ref, cnt0_ref, sem, *, V, T):
    i = pl.program_id(0)
    j = pl.program_id(1)
    neg_inf = jnp.float32(-jnp.inf)

    @pl.when(j == 0)
    def _prologue():
        tv = tv_ref[0]

        @pl.when(i == 0)
        def _first():
            row = jax.lax.broadcasted_iota(jnp.int32, (_B, _TV), 0)
            lane = jax.lax.broadcasted_iota(jnp.int32, (_B, _TV), 1)
            cnt0_ref[...] = row * V + lane

            def loop0(b, _):
                x = xinit_ref[b]
                xs_ref[b, 0] = x
                h_ref[pl.ds(b, 1), :] = emb_ref[pl.ds(x, 1), :] + tv
                return _
            jax.lax.fori_loop(0, _B, loop0, None)

        @pl.when(i > 0)
        def _update():
            def loop(b, _):
                xp = xs_ref[b, 0]
                x1 = x1s_ref[b, 0]
                jump = jnp.logical_and(ub_ref[i - 1, b] < thrb_ref[i - 1],
                                       x1 != xp)
                x = jnp.where(jump, x1, xp)
                xs_ref[b, 0] = x
                h_ref[pl.ds(b, 1), :] = emb_ref[pl.ds(x, 1), :] + tv
                return _
            jax.lax.fori_loop(0, _B, loop, None)

        mx_ref[...] = jnp.full((_B, 1), neg_inf, jnp.float32)
        se_ref[...] = jnp.zeros((_B, 1), jnp.float32)
        s1_ref[...] = jnp.full((_B, 1), neg_inf, jnp.float32)
        s2_ref[...] = jnp.full((_B, 1), neg_inf, jnp.float32)
        l1_ref[...] = jnp.zeros((_B, 1), jnp.float32)
        l2_ref[...] = jnp.zeros((_B, 1), jnp.float32)
        i1_ref[...] = jnp.zeros((_B, 1), jnp.int32)
        i2_ref[...] = jnp.zeros((_B, 1), jnp.int32)

    @pl.when(j < T)
    def _sweep():
        logits = jnp.dot(h_ref[...], w_ref[...],
                         preferred_element_type=jnp.float32) + b_ref[...]

        tmax = jnp.max(logits, axis=1, keepdims=True)
        old_mx = mx_ref[...]
        new_mx = jnp.maximum(old_mx, tmax)
        e = jnp.exp(logits - new_mx)
        se_ref[...] = se_ref[...] * jnp.exp(old_mx - new_mx) \
            + jnp.sum(e, axis=1, keepdims=True)
        mx_ref[...] = new_mx

        cnt0 = cnt0_ref[...]
        cnt = (cnt0 + j * _TV).astype(jnp.uint32)
        g = _gumbel_at(keys_ref[i, 0], keys_ref[i, 1], cnt)

        s = g + logits
        rowv = jax.lax.broadcasted_iota(jnp.int32, (_B, 1), 0) * V
        m1 = jnp.max(s, axis=1, keepdims=True)
        a1 = jnp.argmax(s, axis=1).astype(jnp.int32).reshape(_B, 1)
        oh1 = cnt0 == rowv + a1
        lt1 = _pick_at(oh1, logits, neg_inf)
        s_m = jnp.where(oh1, neg_inf, s)
        m2 = jnp.max(s_m, axis=1, keepdims=True)
        a2 = jnp.argmax(s_m, axis=1).astype(jnp.int32).reshape(_B, 1)
        lt2 = _pick_at(cnt0 == rowv + a2, logits, neg_inf)
        ga1 = a1 + j * _TV
        ga2 = a2 + j * _TV

        rs1, rs2 = s1_ref[...], s2_ref[...]
        take1 = m1 > rs1
        n1_s = jnp.where(take1, m1, rs1)
        n1_l = jnp.where(take1, lt1, l1_ref[...])
        n1_i = jnp.where(take1, ga1, i1_ref[...])
        c_s = jnp.where(take1, rs1, rs2)
        c_l = jnp.where(take1, l1_ref[...], l2_ref[...])
        c_i = jnp.where(take1, i1_ref[...], i2_ref[...])
        d_s = jnp.where(take1, m2, m1)
        d_l = jnp.where(take1, lt2, lt1)
        d_i = jnp.where(take1, ga2, ga1)
        take2 = d_s > c_s
        s2_ref[...] = jnp.where(take2, d_s, c_s)
        l2_ref[...] = jnp.where(take2, d_l, c_l)
        i2_ref[...] = jnp.where(take2, d_i, c_i)
        s1_ref[...] = n1_s
        l1_ref[...] = n1_l
        i1_ref[...] = n1_i

    @pl.when(j == T)
    def _finalize():
        mx = mx_ref[...]
        se = se_ref[...]
        eps = jnp.float32(1e-30)
        rowv = jax.lax.broadcasted_iota(jnp.int32, (_B, 1), 0)
        i1 = i1_ref[...]
        i2 = i2_ref[...]
        k0 = keys_ref[i, 0]
        k1 = keys_ref[i, 1]
        g1 = _gumbel_at(k0, k1, (rowv * V + i1).astype(jnp.uint32))
        g2 = _gumbel_at(k0, k1, (rowv * V + i2).astype(jnp.uint32))
        v1 = jnp.log(jnp.exp(l1_ref[...] - mx) / se + eps) + g1
        v2 = jnp.log(jnp.exp(l2_ref[...] - mx) / se + eps) + g2
        pick2 = (v2 > v1) | ((v2 == v1) & (i2 < i1))
        x1 = jnp.where(pick2, i2, i1)

        @pl.when(i < _N - 1)
        def _handoff():
            x1v_ref[...] = x1
            dma = pltpu.make_async_copy(x1v_ref, x1s_ref, sem)
            dma.start()
            dma.wait()

        @pl.when(i == _N - 1)
        def _emit():
            out_ref[...] = x1


def kernel(x_init, emb, W, b, t_w, n_steps):
    B, S = x_init.shape
    V, D = emb.shape
    step_size = 1.0 / _N
    t_disc = jnp.array([step_size * i for i in range(_N)] + [1.0],
                       dtype=jnp.float32)
    t_disc = t_disc * (n_steps / _N)

    T = -(-V // _TV)
    VP = T * _TV
    w_pad = jnp.pad(W, ((0, 0), (0, VP - V)))
    b_pad = jnp.pad(b, (0, VP - V),
                    constant_values=-jnp.inf).reshape(1, VP)

    key = jax.random.key(42)
    keys = []
    unifs = []
    thrs = []
    for i in range(_N):
        key, k_cat, k_jump, k_cat2 = jax.random.split(key, 4)
        keys.append(jax.random.key_data(k_cat))
        if i < _N - 1:
            t = t_disc[i]
            h = t_disc[i + 1] - t_disc[i]
            unifs.append(jax.random.uniform(k_jump, (B, S)).reshape(B))
            intensity = jnp.float32(1.0) / (1.0 - t)
            thrs.append(1.0 - jnp.exp(-h * intensity))
    keys = jnp.stack(keys)
    ub = jax.lax.bitcast_convert_type(
        jnp.stack(unifs + [jnp.zeros(B, jnp.float32)]), jnp.int32)
    thrb = jax.lax.bitcast_convert_type(
        jnp.stack(thrs + [jnp.float32(0.0)]), jnp.int32)
    tvec = (t_disc[:_N, None] * t_w[None, :]).reshape(_N, 1, D)

    f32 = jnp.float32
    out = pl.pallas_call(
        functools.partial(_body, V=V, T=T),
        grid=(_N, T + 1),
        in_specs=[
            pl.BlockSpec(memory_space=pltpu.SMEM),
            pl.BlockSpec(memory_space=pltpu.SMEM),
            pl.BlockSpec(memory_space=pltpu.SMEM),
            pl.BlockSpec(memory_space=pltpu.SMEM),
            pl.BlockSpec((V, D), lambda i, j: (0, 0)),
            pl.BlockSpec((D, _TV),
                         lambda i, j: (0, jnp.minimum(j, T - 1))),
            pl.BlockSpec((1, _TV),
                         lambda i, j: (0, jnp.minimum(j, T - 1))),
            pl.BlockSpec((1, 1, D), lambda i, j: (i, 0, 0)),
        ],
        out_specs=pl.BlockSpec((_B, 1), lambda i, j: (0, 0)),
        out_shape=jax.ShapeDtypeStruct((_B, 1), jnp.int32),
        scratch_shapes=[
            pltpu.VMEM((_B, 1), f32), pltpu.VMEM((_B, 1), f32),
            pltpu.VMEM((_B, 1), f32), pltpu.VMEM((_B, 1), f32),
            pltpu.VMEM((_B, 1), jnp.int32),
            pltpu.VMEM((_B, 1), f32), pltpu.VMEM((_B, 1), f32),
            pltpu.VMEM((_B, 1), jnp.int32),
            pltpu.VMEM((_B, D), f32),
            pltpu.SMEM((_B, 1), jnp.int32),
            pltpu.VMEM((_B, 1), jnp.int32),
            pltpu.SMEM((_B, 1), jnp.int32),
            pltpu.VMEM((_B, _TV), jnp.int32),
            pltpu.SemaphoreType.DMA,
        ],
        compiler_params=pltpu.CompilerParams(
            dimension_semantics=("arbitrary", "arbitrary"),
            vmem_limit_bytes=60 * 1024 * 1024),
    )(x_init[:, 0], ub, thrb, keys, emb, w_pad, b_pad, tvec)
    return out


# speedup vs baseline: 1.9928x; 1.0287x over previous
import functools

import numpy as np
import jax
import jax.numpy as jnp
from jax.experimental import pallas as pl
from jax.experimental.pallas import tpu as pltpu

_TV = 7168
_B = 128
_N = 10

_TINY = np.float32(np.finfo(np.float32).tiny)


def _threefry2x32(k0, k1, x0, x1):
    def rotl(x, r):
        return (x << jnp.uint32(r)) | (x >> jnp.uint32(32 - r))

    ks2 = k0 ^ k1 ^ jnp.uint32(0x1BD11BDA)
    ks = (k0, k1, ks2)
    rots = ((13, 15, 26, 6), (17, 29, 16, 24))
    x0 = x0 + ks[0]
    x1 = x1 + ks[1]
    for i in range(5):
        for r in rots[i % 2]:
            x0 = x0 + x1
            x1 = rotl(x1, r)
            x1 = x1 ^ x0
        x0 = x0 + ks[(i + 1) % 3]
        x1 = x1 + ks[(i + 2) % 3] + jnp.uint32(i + 1)
    return x0, x1


def _gumbel_at(k0, k1, cnt_u32):
    o0, o1 = _threefry2x32(k0, k1, jnp.uint32(0), cnt_u32)
    bits = o0 ^ o1
    fb = (bits >> jnp.uint32(9)) | jnp.uint32(0x3F800000)
    f = jax.lax.bitcast_convert_type(fb, jnp.float32) - jnp.float32(1.0)
    u = jnp.maximum(_TINY, f + _TINY)
    return -jnp.log(-jnp.log(u))


def _pick_at(onehot, arr, neg_inf):
    return jnp.max(jnp.where(onehot, arr, neg_inf), axis=1, keepdims=True)


def _body(xinit_ref, ub_ref, thrb_ref, keys_ref,
          emb_ref, w_ref, b_ref, tv_ref,
          out_ref,
          mx_ref, se_ref, s1_ref, l1_ref, i1_ref, s2_ref, l2_ref, i2_ref,
          h_ref, xs_ref, x1v_ref, x1s_ref, cnt0_ref, sem, *, V, T):
    i = pl.program_id(0)
    j = pl.program_id(1)
    neg_inf = jnp.float32(-jnp.inf)

    @pl.when(j == 0)
    def _prologue():
        tv = tv_ref[0]

        @pl.when(i == 0)
        def _first():
            row = jax.lax.broadcasted_iota(jnp.int32, (_B, _TV), 0)
            lane = jax.lax.broadcasted_iota(jnp.int32, (_B, _TV), 1)
            cnt0_ref[...] = row * V + lane

            def loop0(b, _):
                xs_ref[b, 0] = xinit_ref[b]
                return _
            jax.lax.fori_loop(0, _B, loop0, None)

        @pl.when(i > 0)
        def _update():
            def loop(b, _):
                xp = xs_ref[b, 0]
                x1 = x1s_ref[b, 0]
                jump = jnp.logical_and(ub_ref[i - 1, b] < thrb_ref[i - 1],
                                       x1 != xp)
                xs_ref[b, 0] = jnp.where(jump, x1, xp)
                return _
            jax.lax.fori_loop(0, _B, loop, None)

        def gather(b, _):
            pltpu.make_async_copy(
                emb_ref.at[pl.ds(xs_ref[b, 0], 1), :],
                h_ref.at[pl.ds(b, 1), :], sem).start()
            return _
        jax.lax.fori_loop(0, _B, gather, None)

        def drain(b, _):
            pltpu.make_async_copy(
                emb_ref.at[pl.ds(0, 1), :],
                h_ref.at[pl.ds(b, 1), :], sem).wait()
            return _
        jax.lax.fori_loop(0, _B, drain, None)
        h_ref[...] = h_ref[...] + tv

        mx_ref[...] = jnp.full((_B, 1), neg_inf, jnp.float32)
        se_ref[...] = jnp.zeros((_B, 1), jnp.float32)
        s1_ref[...] = jnp.full((_B, 1), neg_inf, jnp.float32)
        s2_ref[...] = jnp.full((_B, 1), neg_inf, jnp.float32)
        l1_ref[...] = jnp.zeros((_B, 1), jnp.float32)
        l2_ref[...] = jnp.zeros((_B, 1), jnp.float32)
        i1_ref[...] = jnp.zeros((_B, 1), jnp.int32)
        i2_ref[...] = jnp.zeros((_B, 1), jnp.int32)

    @pl.when(j < T)
    def _sweep():
        logits = jnp.dot(h_ref[...], w_ref[...],
                         preferred_element_type=jnp.float32) + b_ref[...]

        tmax = jnp.max(logits, axis=1, keepdims=True)
        old_mx = mx_ref[...]
        new_mx = jnp.maximum(old_mx, tmax)
        e = jnp.exp(logits - new_mx)
        se_ref[...] = se_ref[...] * jnp.exp(old_mx - new_mx) \
            + jnp.sum(e, axis=1, keepdims=True)
        mx_ref[...] = new_mx

        cnt0 = cnt0_ref[...]
        cnt = (cnt0 + j * _TV).astype(jnp.uint32)
        g = _gumbel_at(keys_ref[i, 0], keys_ref[i, 1], cnt)

        s = g + logits
        rowv = jax.lax.broadcasted_iota(jnp.int32, (_B, 1), 0) * V
        m1 = jnp.max(s, axis=1, keepdims=True)
        a1 = jnp.argmax(s, axis=1).astype(jnp.int32).reshape(_B, 1)
        oh1 = cnt0 == rowv + a1
        lt1 = _pick_at(oh1, logits, neg_inf)
        s_m = jnp.where(oh1, neg_inf, s)
        m2 = jnp.max(s_m, axis=1, keepdims=True)
        a2 = jnp.argmax(s_m, axis=1).astype(jnp.int32).reshape(_B, 1)
        lt2 = _pick_at(cnt0 == rowv + a2, logits, neg_inf)
        ga1 = a1 + j * _TV
        ga2 = a2 + j * _TV

        rs1, rs2 = s1_ref[...], s2_ref[...]
        take1 = m1 > rs1
        n1_s = jnp.where(take1, m1, rs1)
        n1_l = jnp.where(take1, lt1, l1_ref[...])
        n1_i = jnp.where(take1, ga1, i1_ref[...])
        c_s = jnp.where(take1, rs1, rs2)
        c_l = jnp.where(take1, l1_ref[...], l2_ref[...])
        c_i = jnp.where(take1, i1_ref[...], i2_ref[...])
        d_s = jnp.where(take1, m2, m1)
        d_l = jnp.where(take1, lt2, lt1)
        d_i = jnp.where(take1, ga2, ga1)
        take2 = d_s > c_s
        s2_ref[...] = jnp.where(take2, d_s, c_s)
        l2_ref[...] = jnp.where(take2, d_l, c_l)
        i2_ref[...] = jnp.where(take2, d_i, c_i)
        s1_ref[...] = n1_s
        l1_ref[...] = n1_l
        i1_ref[...] = n1_i

    @pl.when(j == T)
    def _finalize():
        mx = mx_ref[...]
        se = se_ref[...]
        eps = jnp.float32(1e-30)
        rowv = jax.lax.broadcasted_iota(jnp.int32, (_B, 1), 0)
        i1 = i1_ref[...]
        i2 = i2_ref[...]
        k0 = keys_ref[i, 0]
        k1 = keys_ref[i, 1]
        g1 = _gumbel_at(k0, k1, (rowv * V + i1).astype(jnp.uint32))
        g2 = _gumbel_at(k0, k1, (rowv * V + i2).astype(jnp.uint32))
        v1 = jnp.log(jnp.exp(l1_ref[...] - mx) / se + eps) + g1
        v2 = jnp.log(jnp.exp(l2_ref[...] - mx) / se + eps) + g2
        pick2 = (v2 > v1) | ((v2 == v1) & (i2 < i1))
        x1 = jnp.where(pick2, i2, i1)

        @pl.when(i < _N - 1)
        def _handoff():
            x1v_ref[...] = x1
            dma = pltpu.make_async_copy(x1v_ref, x1s_ref, sem)
            dma.start()
            dma.wait()

        @pl.when(i == _N - 1)
        def _emit():
            out_ref[...] = x1


def kernel(x_init, emb, W, b, t_w, n_steps):
    B, S = x_init.shape
    V, D = emb.shape
    step_size = 1.0 / _N
    t_disc = jnp.array([step_size * i for i in range(_N)] + [1.0],
                       dtype=jnp.float32)
    t_disc = t_disc * (n_steps / _N)

    T = -(-V // _TV)
    VP = T * _TV
    w_pad = jnp.pad(W, ((0, 0), (0, VP - V)))
    b_pad = jnp.pad(b, (0, VP - V),
                    constant_values=-jnp.inf).reshape(1, VP)

    key = jax.random.key(42)
    keys = []
    unifs = []
    thrs = []
    for i in range(_N):
        key, k_cat, k_jump, k_cat2 = jax.random.split(key, 4)
        keys.append(jax.random.key_data(k_cat))
        if i < _N - 1:
            t = t_disc[i]
            h = t_disc[i + 1] - t_disc[i]
            unifs.append(jax.random.uniform(k_jump, (B, S)).reshape(B))
            intensity = jnp.float32(1.0) / (1.0 - t)
            thrs.append(1.0 - jnp.exp(-h * intensity))
    keys = jnp.stack(keys)
    ub = jax.lax.bitcast_convert_type(
        jnp.stack(unifs + [jnp.zeros(B, jnp.float32)]), jnp.int32)
    thrb = jax.lax.bitcast_convert_type(
        jnp.stack(thrs + [jnp.float32(0.0)]), jnp.int32)
    tvec = (t_disc[:_N, None] * t_w[None, :]).reshape(_N, 1, D)

    f32 = jnp.float32
    out = pl.pallas_call(
        functools.partial(_body, V=V, T=T),
        grid=(_N, T + 1),
        in_specs=[
            pl.BlockSpec(memory_space=pltpu.SMEM),
            pl.BlockSpec(memory_space=pltpu.SMEM),
            pl.BlockSpec(memory_space=pltpu.SMEM),
            pl.BlockSpec(memory_space=pltpu.SMEM),
            pl.BlockSpec(memory_space=pltpu.MemorySpace.HBM),
            pl.BlockSpec((D, _TV),
                         lambda i, j: (0, jnp.minimum(j, T - 1))),
            pl.BlockSpec((1, _TV),
                         lambda i, j: (0, jnp.minimum(j, T - 1))),
            pl.BlockSpec((1, 1, D), lambda i, j: (i, 0, 0)),
        ],
        out_specs=pl.BlockSpec((_B, 1), lambda i, j: (0, 0)),
        out_shape=jax.ShapeDtypeStruct((_B, 1), jnp.int32),
        scratch_shapes=[
            pltpu.VMEM((_B, 1), f32), pltpu.VMEM((_B, 1), f32),
            pltpu.VMEM((_B, 1), f32), pltpu.VMEM((_B, 1), f32),
            pltpu.VMEM((_B, 1), jnp.int32),
            pltpu.VMEM((_B, 1), f32), pltpu.VMEM((_B, 1), f32),
            pltpu.VMEM((_B, 1), jnp.int32),
            pltpu.VMEM((_B, D), f32),
            pltpu.SMEM((_B, 1), jnp.int32),
            pltpu.VMEM((_B, 1), jnp.int32),
            pltpu.SMEM((_B, 1), jnp.int32),
            pltpu.VMEM((_B, _TV), jnp.int32),
            pltpu.SemaphoreType.DMA,
        ],
        compiler_params=pltpu.CompilerParams(
            dimension_semantics=("arbitrary", "arbitrary"),
            vmem_limit_bytes=60 * 1024 * 1024),
    )(x_init[:, 0], ub, thrb, keys, emb, w_pad, b_pad, tvec)
    return out
